# Initial kernel scaffold; baseline (speedup 1.0000x reference)
import functools

import jax
import jax.numpy as jnp
from jax import lax
from jax.experimental import pallas as pl
from jax.experimental.pallas import tpu as pltpu
from jax.experimental.pallas import tpu_sc as plsc

_NEG = jnp.float32(-3.0e38)
_K = 20


def _edge_call(xt, Wp, tp, Wq, blk=512):
    B, N, C = xt.shape
    Cout = Wp.shape[0]

    def body(xblk_ref, xall_ref, wp_ref, tp_ref, wq_ref, idx_ref, pt_ref, qt_ref):
        b = pl.program_id(0)
        xb = xblk_ref[0]
        xa = xall_ref[0]
        dot = lax.dot_general(xb, xa, (((1,), (1,)), ((), ())),
                              preferred_element_type=jnp.float32)
        xx = jnp.sum(xa * xa, axis=1)
        s = 2.0 * dot - xx[None, :]
        iota = lax.broadcasted_iota(jnp.int32, (blk, N), 1)
        cols = []
        for _ in range(_K):
            m = jnp.max(s, axis=1, keepdims=True)
            eq = s == m
            cols.append(jnp.min(jnp.where(eq, iota, N), axis=1))
            s = jnp.where(eq, _NEG, s)
        idx_ref[0] = jnp.stack(cols, axis=1) + b * N
        pt_ref[0] = lax.dot_general(xb, wp_ref[...], (((1,), (1,)), ((), ())),
                                    preferred_element_type=jnp.float32) + tp_ref[...]
        qt_ref[0] = lax.dot_general(xb, wq_ref[...], (((1,), (1,)), ((), ())),
                                    preferred_element_type=jnp.float32)

    return pl.pallas_call(
        body,
        grid=(B, N // blk),
        in_specs=[
            pl.BlockSpec((1, blk, C), lambda b, n: (b, n, 0)),
            pl.BlockSpec((1, N, C), lambda b, n: (b, 0, 0)),
            pl.BlockSpec((Cout, C), lambda b, n: (0, 0)),
            pl.BlockSpec((1, Cout), lambda b, n: (0, 0)),
            pl.BlockSpec((Cout, C), lambda b, n: (0, 0)),
        ],
        out_specs=[
            pl.BlockSpec((1, blk, _K), lambda b, n: (b, n, 0)),
            pl.BlockSpec((1, blk, Cout), lambda b, n: (b, n, 0)),
            pl.BlockSpec((1, blk, Cout), lambda b, n: (b, n, 0)),
        ],
        out_shape=[
            jax.ShapeDtypeStruct((B, N, _K), jnp.int32),
            jax.ShapeDtypeStruct((B, N, Cout), jnp.float32),
            jax.ShapeDtypeStruct((B, N, Cout), jnp.float32),
        ],
    )(xt, xt, Wp, tp.reshape(1, -1), Wq)


def _gather_max_sc(qt, idx2, pt):
    R, Cout = qt.shape
    NW = 32
    PW = R // NW
    PC = 16
    NCH = PW // PC
    NSUB = (PC * _K) // 80
    G = Cout // 16

    @functools.partial(
        pl.kernel,
        mesh=plsc.VectorSubcoreMesh(core_axis_name="c", subcore_axis_name="s"),
        out_type=jax.ShapeDtypeStruct((R, Cout), jnp.float32),
        scratch_types=[
            pltpu.VMEM((NSUB, 80), jnp.int32),
            pltpu.VMEM((PC * _K, Cout), jnp.float32),
            pltpu.VMEM((PC, Cout), jnp.float32),
            pltpu.VMEM((PC, Cout), jnp.float32),
            pltpu.SemaphoreType.DMA,
        ],
    )
    def k(qt_hbm, idx_hbm, pt_hbm, out_hbm, idx_v, rows_v, pt_v, out_v, sem):
        wid = lax.axis_index("s") * 2 + lax.axis_index("c")

        def chunk(c, carry):
            base = wid * PW + c * PC
            pltpu.sync_copy(idx_hbm.at[pl.ds((base * _K) // 80, NSUB)], idx_v)
            cps = [
                pltpu.async_copy(qt_hbm.at[idx_v.at[d]],
                                 rows_v.at[pl.ds(d * 80, 80)], sem)
                for d in range(NSUB)
            ]
            for cp in cps:
                cp.wait()
            pltpu.sync_copy(pt_hbm.at[pl.ds(base, PC)], pt_v)

            def point(p, carry2):
                r0 = p * _K
                for g in range(G):
                    sl = pl.ds(g * 16, 16)
                    acc = rows_v[r0, sl]
                    for j in range(1, _K):
                        acc = jnp.maximum(acc, rows_v[r0 + j, sl])
                    y = acc + pt_v[p, sl]
                    out_v[p, sl] = jnp.where(y >= 0.0, y, 0.2 * y)
                return carry2

            lax.fori_loop(0, PC, point, 0)
            pltpu.sync_copy(out_v, out_hbm.at[pl.ds(base, PC)])
            return carry

        lax.fori_loop(0, NCH, chunk, 0)

    return k(qt, idx2, pt)


def _head_global(x1, x2, x3, x4, W5s, t5, blk=512):
    B, N, _ = x1.shape
    Cg = W5s.shape[0]

    def body(x1r, x2r, x3r, x4r, wr, tr, outr):
        n = pl.program_id(1)
        xc = jnp.concatenate([x1r[0], x2r[0], x3r[0], x4r[0]], axis=1)
        xg = lax.dot_general(xc, wr[...], (((1,), (1,)), ((), ())),
                             preferred_element_type=jnp.float32) + tr[...]
        xg = jnp.where(xg >= 0.0, xg, 0.2 * xg)
        m = jnp.max(xg, axis=0, keepdims=True)

        @pl.when(n == 0)
        def _():
            outr[...] = m

        @pl.when(n != 0)
        def _():
            outr[...] = jnp.maximum(outr[...], m)

    ins = [pl.BlockSpec((1, blk, a.shape[2]), lambda b, n: (b, n, 0))
           for a in (x1, x2, x3, x4)]
    return pl.pallas_call(
        body,
        grid=(B, N // blk),
        in_specs=ins + [
            pl.BlockSpec(W5s.shape, lambda b, n: (0, 0)),
            pl.BlockSpec((1, Cg), lambda b, n: (0, 0)),
        ],
        out_specs=pl.BlockSpec((1, Cg), lambda b, n: (b, 0)),
        out_shape=jax.ShapeDtypeStruct((B, Cg), jnp.float32),
    )(x1, x2, x3, x4, W5s, t5.reshape(1, -1))


def _head_seg(x1, x2, x3, x4, xglob, Wh1a, Wh1b, th1, Wh2s, th2, Wh3, blk=512):
    B, N, _ = x1.shape
    NC = Wh3.shape[0]

    def body(x1r, x2r, x3r, x4r, gr, w1ar, w1br, t1r, w2r, t2r, w3r, outr):
        xc = jnp.concatenate([x1r[0], x2r[0], x3r[0], x4r[0]], axis=1)
        gterm = lax.dot_general(gr[...], w1br[...], (((1,), (1,)), ((), ())),
                                preferred_element_type=jnp.float32)
        h = lax.dot_general(xc, w1ar[...], (((1,), (1,)), ((), ())),
                            preferred_element_type=jnp.float32) + gterm + t1r[...]
        h = jnp.where(h >= 0.0, h, 0.2 * h)
        h = lax.dot_general(h, w2r[...], (((1,), (1,)), ((), ())),
                            preferred_element_type=jnp.float32) + t2r[...]
        h = jnp.where(h >= 0.0, h, 0.2 * h)
        outr[0] = lax.dot_general(h, w3r[...], (((1,), (1,)), ((), ())),
                                  preferred_element_type=jnp.float32)

    ins = [pl.BlockSpec((1, blk, a.shape[2]), lambda b, n: (b, n, 0))
           for a in (x1, x2, x3, x4)]
    return pl.pallas_call(
        body,
        grid=(B, N // blk),
        in_specs=ins + [
            pl.BlockSpec((1, xglob.shape[1]), lambda b, n: (b, 0)),
            pl.BlockSpec(Wh1a.shape, lambda b, n: (0, 0)),
            pl.BlockSpec(Wh1b.shape, lambda b, n: (0, 0)),
            pl.BlockSpec((1, th1.shape[0]), lambda b, n: (0, 0)),
            pl.BlockSpec(Wh2s.shape, lambda b, n: (0, 0)),
            pl.BlockSpec((1, th2.shape[0]), lambda b, n: (0, 0)),
            pl.BlockSpec(Wh3.shape, lambda b, n: (0, 0)),
        ],
        out_specs=pl.BlockSpec((1, blk, NC), lambda b, n: (b, n, 0)),
        out_shape=jax.ShapeDtypeStruct((B, N, NC), jnp.float32),
    )(x1, x2, x3, x4, xglob, Wh1a, Wh1b, th1.reshape(1, -1), Wh2s,
      th2.reshape(1, -1), Wh3)


def _fold(W, g, b, m, v, cin):
    s = g / jnp.sqrt(v + 1e-5)
    t = b - m * s
    Wa, Wb = W[:, :cin], W[:, cin:]
    return (Wa - Wb) * s[:, None], t, Wb * s[:, None]


def _edge_layer(xt, W, g, b, m, v):
    B, N, C = xt.shape
    Wp, tp, Wq = _fold(W, g, b, m, v, C)
    idx, PT, QT = _edge_call(xt, Wp, tp, Wq)
    R = B * N
    Cout = Wp.shape[0]
    out = _gather_max_sc(QT.reshape(R, Cout),
                         idx.reshape((R * _K) // 80, 80),
                         PT.reshape(R, Cout))
    return out.reshape(B, N, Cout)


def kernel(x, W1, g1, b1, m1, v1, W2, g2, b2, m2, v2, W3, g3, b3, m3, v3,
           W4, g4, b4, m4, v4, W5, g5, b5, m5, v5, Wh1, gh1, bh1, mh1, vh1,
           Wh2, gh2, bh2, mh2, vh2, Wh3):
    B, N, F = x.shape
    x0 = jnp.pad(x, ((0, 0), (0, 0), (0, 16 - F)))
    W1p = jnp.concatenate(
        [W1[:, :F], jnp.zeros((W1.shape[0], 16 - F), W1.dtype),
         W1[:, F:], jnp.zeros((W1.shape[0], 16 - F), W1.dtype)], axis=1)

    x1 = _edge_layer(x0, W1p, g1, b1, m1, v1)
    x2 = _edge_layer(x1, W2, g2, b2, m2, v2)
    x3 = _edge_layer(x2, W3, g3, b3, m3, v3)
    x4 = _edge_layer(x3, W4, g4, b4, m4, v4)

    s5 = g5 / jnp.sqrt(v5 + 1e-5)
    t5 = b5 - m5 * s5
    xglob = _head_global(x1, x2, x3, x4, W5 * s5[:, None], t5)

    sh1 = gh1 / jnp.sqrt(vh1 + 1e-5)
    th1 = bh1 - mh1 * sh1
    Wh1s = Wh1 * sh1[:, None]
    sh2 = gh2 / jnp.sqrt(vh2 + 1e-5)
    th2 = bh2 - mh2 * sh2
    return _head_seg(x1, x2, x3, x4, xglob, Wh1s[:, :512], Wh1s[:, 512:],
                     th1, Wh2 * sh2[:, None], th2, Wh3)


# speedup vs baseline: 5.1793x; 5.1793x over previous
import functools

import jax
import jax.numpy as jnp
from jax import lax
from jax.experimental import pallas as pl
from jax.experimental.pallas import tpu as pltpu
from jax.experimental.pallas import tpu_sc as plsc

_NEG = -3.0e38
_K = 20
_BF = jnp.bfloat16


def _mm(a, b):
    return lax.dot_general(a.astype(_BF), b.astype(_BF),
                           (((1,), (1,)), ((), ())),
                           preferred_element_type=jnp.float32)


def _bn_lrelu(y, g, b, m, v):
    t = (y - m) / jnp.sqrt(v + 1e-5) * g + b
    return jnp.where(t >= 0.0, t, 0.2 * t)


def _knn_call(xt, blk=512):
    B, N, C = xt.shape

    def body(xblk_ref, xall_ref, idx_ref):
        b = pl.program_id(0)
        xb = xblk_ref[0]
        xa = xall_ref[0]
        dot = _mm(xb, xa)
        xxr = jnp.sum(xb * xb, axis=1)
        xxc = jnp.sum(xa * xa, axis=1)
        s = (2.0 * dot - xxr[:, None]) - xxc[None, :]
        iota = lax.broadcasted_iota(jnp.int32, (blk, N), 1)
        cols = []
        for _ in range(_K):
            m = jnp.max(s, axis=1, keepdims=True)
            eq = s == m
            cols.append(jnp.min(jnp.where(eq, iota, N), axis=1))
            s = jnp.where(eq, _NEG, s)
        idx_ref[0] = jnp.stack(cols, axis=1) + b * N

    return pl.pallas_call(
        body,
        grid=(B, N // blk),
        in_specs=[
            pl.BlockSpec((1, blk, C), lambda b, n: (b, n, 0)),
            pl.BlockSpec((1, N, C), lambda b, n: (b, 0, 0)),
        ],
        out_specs=pl.BlockSpec((1, blk, _K), lambda b, n: (b, n, 0)),
        out_shape=jax.ShapeDtypeStruct((B, N, _K), jnp.int32),
    )(xt, xt)


def _gather_diff_sc(xp, idx3):
    R, Cp = xp.shape
    NW = 32
    PW = R // NW
    PC = 16 if Cp <= 128 else 8
    NCH = PW // PC
    NSUB = (PC * _K) // 80
    G = Cp // 16

    @functools.partial(
        pl.kernel,
        mesh=plsc.VectorSubcoreMesh(core_axis_name="c", subcore_axis_name="s"),
        out_type=jax.ShapeDtypeStruct((_K, R, Cp), jnp.float32),
        scratch_types=[
            pltpu.VMEM(((PW * _K) // 80, 80), jnp.int32),
            pltpu.VMEM((PC * _K, Cp), jnp.float32),
            pltpu.VMEM((PC, Cp), jnp.float32),
            pltpu.VMEM((_K, PC, Cp), jnp.float32),
            pltpu.SemaphoreType.DMA,
        ],
    )
    def k(xp_hbm, idx_hbm, out_hbm, idx_v, rows_v, xi_v, out_v, sem):
        wid = lax.axis_index("s") * 2 + lax.axis_index("c")
        pltpu.sync_copy(idx_hbm.at[wid], idx_v)

        def chunk(c, carry):
            base = wid * PW + c * PC
            cps = [
                pltpu.async_copy(xp_hbm.at[idx_v.at[c * NSUB + d]],
                                 rows_v.at[pl.ds(d * 80, 80)], sem)
                for d in range(NSUB)
            ]
            for cp in cps:
                cp.wait()
            pltpu.sync_copy(xp_hbm.at[pl.ds(base, PC)], xi_v)

            def point(p, carry2):
                r0 = p * _K
                for g in range(G):
                    sl = pl.ds(g * 16, 16)
                    xi = xi_v[p, sl]
                    for j in range(_K):
                        out_v[j, p, sl] = rows_v[r0 + j, sl] - xi
                return carry2

            lax.fori_loop(0, PC, point, 0)
            for j in range(_K):
                pltpu.sync_copy(out_v.at[j], out_hbm.at[j, pl.ds(base, PC)])
            return carry

        lax.fori_loop(0, NCH, chunk, 0)

    return k(xp, idx3)


def _edgeconv_call(xp, diff, W, g, b, m, v, C, cpn, blkp=128):
    R, Cp = xp.shape
    Cout = W.shape[0]
    Wa, Wb = W[:, :C], W[:, C:]

    def body(x_ref, d_ref, wa_ref, wb_ref, g_ref, b_ref, m_ref, v_ref,
             out_ref, pad_ref):
        xb = x_ref[...][:, :C]
        y1 = _mm(xb, wa_ref[...])
        d2 = d_ref[...].reshape(_K * blkp, Cp)[:, :C]
        y2 = _mm(d2, wb_ref[...]).reshape(_K, blkp, Cout)
        M = jnp.max(y2, axis=0)
        t = _bn_lrelu(y1 + M, g_ref[...], b_ref[...], m_ref[...], v_ref[...])
        out_ref[...] = t
        pad_ref[...] = jnp.concatenate(
            [t, jnp.zeros((blkp, cpn - Cout), jnp.float32)], axis=1) \
            if cpn > Cout else t

    return pl.pallas_call(
        body,
        grid=(R // blkp,),
        in_specs=[
            pl.BlockSpec((blkp, Cp), lambda r: (r, 0)),
            pl.BlockSpec((_K, blkp, Cp), lambda r: (0, r, 0)),
            pl.BlockSpec((Cout, C), lambda r: (0, 0)),
            pl.BlockSpec((Cout, C), lambda r: (0, 0)),
            pl.BlockSpec((1, Cout), lambda r: (0, 0)),
            pl.BlockSpec((1, Cout), lambda r: (0, 0)),
            pl.BlockSpec((1, Cout), lambda r: (0, 0)),
            pl.BlockSpec((1, Cout), lambda r: (0, 0)),
        ],
        out_specs=[
            pl.BlockSpec((blkp, Cout), lambda r: (r, 0)),
            pl.BlockSpec((blkp, cpn), lambda r: (r, 0)),
        ],
        out_shape=[
            jax.ShapeDtypeStruct((R, Cout), jnp.float32),
            jax.ShapeDtypeStruct((R, cpn), jnp.float32),
        ],
    )(xp, diff, Wa, Wb, g.reshape(1, -1), b.reshape(1, -1),
      m.reshape(1, -1), v.reshape(1, -1))


def _head_global(x1, x2, x3, x4, W5, g5, b5, m5, v5, blk=512):
    B, N, _ = x1.shape
    Cg = W5.shape[0]

    def body(x1r, x2r, x3r, x4r, wr, gr, br, mr, vr, outr):
        n = pl.program_id(1)
        xc = jnp.concatenate([x1r[0], x2r[0], x3r[0], x4r[0]], axis=1)
        xg = _bn_lrelu(_mm(xc, wr[...]), gr[...], br[...], mr[...], vr[...])
        m = jnp.max(xg, axis=0, keepdims=True)[None]

        @pl.when(n == 0)
        def _():
            outr[...] = m

        @pl.when(n != 0)
        def _():
            outr[...] = jnp.maximum(outr[...], m)

    ins = [pl.BlockSpec((1, blk, a.shape[2]), lambda b, n: (b, n, 0))
           for a in (x1, x2, x3, x4)]
    return pl.pallas_call(
        body,
        grid=(B, N // blk),
        in_specs=ins + [pl.BlockSpec(W5.shape, lambda b, n: (0, 0))] +
        [pl.BlockSpec((1, Cg), lambda b, n: (0, 0))] * 4,
        out_specs=pl.BlockSpec((1, 1, Cg), lambda b, n: (b, 0, 0)),
        out_shape=jax.ShapeDtypeStruct((B, 1, Cg), jnp.float32),
    )(x1, x2, x3, x4, W5, g5.reshape(1, -1), b5.reshape(1, -1),
      m5.reshape(1, -1), v5.reshape(1, -1))


def _head_seg(x1, x2, x3, x4, xglob, Wh1a, Wh1b, gh1, bh1, mh1, vh1,
              Wh2, gh2, bh2, mh2, vh2, Wh3, blk=512):
    B, N, _ = x1.shape
    NC = Wh3.shape[0]

    def body(x1r, x2r, x3r, x4r, gr, w1ar, w1br, g1r, b1r, m1r, v1r,
             w2r, g2r, b2r, m2r, v2r, w3r, outr):
        xc = jnp.concatenate([x1r[0], x2r[0], x3r[0], x4r[0]], axis=1)
        h = _mm(xc, w1ar[...]) + _mm(gr[0], w1br[...])
        h = _bn_lrelu(h, g1r[...], b1r[...], m1r[...], v1r[...])
        h = _bn_lrelu(_mm(h, w2r[...]), g2r[...], b2r[...], m2r[...], v2r[...])
        outr[0] = _mm(h, w3r[...])

    ins = [pl.BlockSpec((1, blk, a.shape[2]), lambda b, n: (b, n, 0))
           for a in (x1, x2, x3, x4)]
    return pl.pallas_call(
        body,
        grid=(B, N // blk),
        in_specs=ins + [
            pl.BlockSpec((1, 1, xglob.shape[2]), lambda b, n: (b, 0, 0)),
            pl.BlockSpec(Wh1a.shape, lambda b, n: (0, 0)),
            pl.BlockSpec(Wh1b.shape, lambda b, n: (0, 0)),
            pl.BlockSpec((1, 512), lambda b, n: (0, 0)),
            pl.BlockSpec((1, 512), lambda b, n: (0, 0)),
            pl.BlockSpec((1, 512), lambda b, n: (0, 0)),
            pl.BlockSpec((1, 512), lambda b, n: (0, 0)),
            pl.BlockSpec(Wh2.shape, lambda b, n: (0, 0)),
            pl.BlockSpec((1, 256), lambda b, n: (0, 0)),
            pl.BlockSpec((1, 256), lambda b, n: (0, 0)),
            pl.BlockSpec((1, 256), lambda b, n: (0, 0)),
            pl.BlockSpec((1, 256), lambda b, n: (0, 0)),
            pl.BlockSpec(Wh3.shape, lambda b, n: (0, 0)),
        ],
        out_specs=pl.BlockSpec((1, blk, NC), lambda b, n: (b, n, 0)),
        out_shape=jax.ShapeDtypeStruct((B, N, NC), jnp.float32),
    )(x1, x2, x3, x4, xglob, Wh1a, Wh1b,
      gh1.reshape(1, -1), bh1.reshape(1, -1), mh1.reshape(1, -1),
      vh1.reshape(1, -1), Wh2, gh2.reshape(1, -1), bh2.reshape(1, -1),
      mh2.reshape(1, -1), vh2.reshape(1, -1), Wh3)


def _edge_layer(xp, W, g, b, m, v, C, cpn):
    R, Cp = xp.shape
    B, N = 4, 2048
    idx = _knn_call(xp.reshape(B, N, Cp))
    diff = _gather_diff_sc(xp, idx.reshape(32, (R * _K) // (32 * 80), 80))
    return _edgeconv_call(xp, diff, W, g, b, m, v, C, cpn)


def kernel(x, W1, g1, b1, m1, v1, W2, g2, b2, m2, v2, W3, g3, b3, m3, v3,
           W4, g4, b4, m4, v4, W5, g5, b5, m5, v5, Wh1, gh1, bh1, mh1, vh1,
           Wh2, gh2, bh2, mh2, vh2, Wh3):
    B, N, F = x.shape
    R = B * N
    xp0 = jnp.pad(x.reshape(R, F), ((0, 0), (0, 128 - F)))

    x1, x1p = _edge_layer(xp0, W1, g1, b1, m1, v1, F, 128)
    x2, x2p = _edge_layer(x1p, W2, g2, b2, m2, v2, 64, 128)
    x3, x3p = _edge_layer(x2p, W3, g3, b3, m3, v3, 64, 128)
    x4, _ = _edge_layer(x3p, W4, g4, b4, m4, v4, 128, 256)

    x1 = x1.reshape(B, N, 64)
    x2 = x2.reshape(B, N, 64)
    x3 = x3.reshape(B, N, 128)
    x4 = x4.reshape(B, N, 256)

    xglob = _head_global(x1, x2, x3, x4, W5, g5, b5, m5, v5)
    return _head_seg(x1, x2, x3, x4, xglob, Wh1[:, :512], Wh1[:, 512:],
                     gh1, bh1, mh1, vh1, Wh2, gh2, bh2, mh2, vh2, Wh3)


# speedup vs baseline: 7.2878x; 1.4071x over previous
import functools

import jax
import jax.numpy as jnp
from jax import lax
from jax.experimental import pallas as pl
from jax.experimental.pallas import tpu as pltpu
from jax.experimental.pallas import tpu_sc as plsc

_NEG = -3.0e38
_K = 20
_BF = jnp.bfloat16


def _mm(a, b):
    return lax.dot_general(a.astype(_BF), b.astype(_BF),
                           (((1,), (1,)), ((), ())),
                           preferred_element_type=jnp.float32)


def _bn_lrelu(y, g, b, m, v):
    t = (y - m) / jnp.sqrt(v + 1e-5) * g + b
    return jnp.where(t >= 0.0, t, 0.2 * t)


def _knn_call(xt, blk=512):
    B, N, C = xt.shape

    def body(xblk_ref, xall_ref, idx_ref):
        b = pl.program_id(0)
        xb = xblk_ref[0]
        xa = xall_ref[0]
        dot = _mm(xb, xa)
        xxr = jnp.sum(xb * xb, axis=1)
        xxc = jnp.sum(xa * xa, axis=1)
        s = (2.0 * dot - xxr[:, None]) - xxc[None, :]
        iota = lax.broadcasted_iota(jnp.int32, (blk, N), 1)
        cols = []
        for _ in range(_K):
            m = jnp.max(s, axis=1, keepdims=True)
            eq = s == m
            cols.append(jnp.min(jnp.where(eq, iota, N), axis=1))
            s = jnp.where(eq, _NEG, s)
        idx_ref[0] = jnp.stack(cols, axis=1) + b * N

    return pl.pallas_call(
        body,
        grid=(B, N // blk),
        in_specs=[
            pl.BlockSpec((1, blk, C), lambda b, n: (b, n, 0)),
            pl.BlockSpec((1, N, C), lambda b, n: (b, 0, 0)),
        ],
        out_specs=pl.BlockSpec((1, blk, _K), lambda b, n: (b, n, 0)),
        out_shape=jax.ShapeDtypeStruct((B, N, _K), jnp.int32),
    )(xt, xt)


def _gather_rows_sc(xp, idx3):
    R, Cp = xp.shape
    NW = 32
    PW = R // NW
    PC = 16
    NCH = PW // PC
    NSUB = (PC * _K) // 80

    @functools.partial(
        pl.kernel,
        mesh=plsc.VectorSubcoreMesh(core_axis_name="c", subcore_axis_name="s"),
        out_type=jax.ShapeDtypeStruct((R * _K, Cp), jnp.float32),
        scratch_types=[
            pltpu.VMEM(((PW * _K) // 80, 80), jnp.int32),
            pltpu.VMEM((PC * _K, Cp), jnp.float32),
            pltpu.SemaphoreType.DMA,
        ],
    )
    def k(xp_hbm, idx_hbm, out_hbm, idx_v, rows_v, sem):
        wid = lax.axis_index("s") * 2 + lax.axis_index("c")
        pltpu.sync_copy(idx_hbm.at[wid], idx_v)

        def chunk(c, carry):
            base = wid * PW + c * PC
            cps = [
                pltpu.async_copy(xp_hbm.at[idx_v.at[c * NSUB + d]],
                                 rows_v.at[pl.ds(d * 80, 80)], sem)
                for d in range(NSUB)
            ]
            for cp in cps:
                cp.wait()
            pltpu.sync_copy(rows_v, out_hbm.at[pl.ds(base * _K, PC * _K)])
            return carry

        lax.fori_loop(0, NCH, chunk, 0)

    return k(xp, idx3)


def _edgeconv_call(xp, diff, W, g, b, m, v, C, cpn, blkp=128):
    R, Cp = xp.shape
    Cout = W.shape[0]
    Wa, Wb = W[:, :C], W[:, C:]

    def body(x_ref, d_ref, wa_ref, wb_ref, g_ref, b_ref, m_ref, v_ref,
             out_ref, pad_ref):
        xb = x_ref[...][:, :C]
        y1 = _mm(xb, wa_ref[...])
        rows = d_ref[...][:, :C]
        xrep = jnp.broadcast_to(xb[:, None, :], (blkp, _K, C)).reshape(
            blkp * _K, C)
        y2 = _mm(rows - xrep, wb_ref[...]).reshape(blkp, _K, Cout)
        M = jnp.max(y2, axis=1)
        t = _bn_lrelu(y1 + M, g_ref[...], b_ref[...], m_ref[...], v_ref[...])
        out_ref[...] = t
        pad_ref[...] = jnp.concatenate(
            [t, jnp.zeros((blkp, cpn - Cout), jnp.float32)], axis=1) \
            if cpn > Cout else t

    return pl.pallas_call(
        body,
        grid=(R // blkp,),
        in_specs=[
            pl.BlockSpec((blkp, Cp), lambda r: (r, 0)),
            pl.BlockSpec((blkp * _K, Cp), lambda r: (r, 0)),
            pl.BlockSpec((Cout, C), lambda r: (0, 0)),
            pl.BlockSpec((Cout, C), lambda r: (0, 0)),
            pl.BlockSpec((1, Cout), lambda r: (0, 0)),
            pl.BlockSpec((1, Cout), lambda r: (0, 0)),
            pl.BlockSpec((1, Cout), lambda r: (0, 0)),
            pl.BlockSpec((1, Cout), lambda r: (0, 0)),
        ],
        out_specs=[
            pl.BlockSpec((blkp, Cout), lambda r: (r, 0)),
            pl.BlockSpec((blkp, cpn), lambda r: (r, 0)),
        ],
        out_shape=[
            jax.ShapeDtypeStruct((R, Cout), jnp.float32),
            jax.ShapeDtypeStruct((R, cpn), jnp.float32),
        ],
    )(xp, diff, Wa, Wb, g.reshape(1, -1), b.reshape(1, -1),
      m.reshape(1, -1), v.reshape(1, -1))


def _head_global(x1, x2, x3, x4, W5, g5, b5, m5, v5, blk=512):
    B, N, _ = x1.shape
    Cg = W5.shape[0]

    def body(x1r, x2r, x3r, x4r, wr, gr, br, mr, vr, outr):
        n = pl.program_id(1)
        xc = jnp.concatenate([x1r[0], x2r[0], x3r[0], x4r[0]], axis=1)
        xg = _bn_lrelu(_mm(xc, wr[...]), gr[...], br[...], mr[...], vr[...])
        m = jnp.max(xg, axis=0, keepdims=True)[None]

        @pl.when(n == 0)
        def _():
            outr[...] = m

        @pl.when(n != 0)
        def _():
            outr[...] = jnp.maximum(outr[...], m)

    ins = [pl.BlockSpec((1, blk, a.shape[2]), lambda b, n: (b, n, 0))
           for a in (x1, x2, x3, x4)]
    return pl.pallas_call(
        body,
        grid=(B, N // blk),
        in_specs=ins + [pl.BlockSpec(W5.shape, lambda b, n: (0, 0))] +
        [pl.BlockSpec((1, Cg), lambda b, n: (0, 0))] * 4,
        out_specs=pl.BlockSpec((1, 1, Cg), lambda b, n: (b, 0, 0)),
        out_shape=jax.ShapeDtypeStruct((B, 1, Cg), jnp.float32),
    )(x1, x2, x3, x4, W5, g5.reshape(1, -1), b5.reshape(1, -1),
      m5.reshape(1, -1), v5.reshape(1, -1))


def _head_seg(x1, x2, x3, x4, xglob, Wh1a, Wh1b, gh1, bh1, mh1, vh1,
              Wh2, gh2, bh2, mh2, vh2, Wh3, blk=512):
    B, N, _ = x1.shape
    NC = Wh3.shape[0]

    def body(x1r, x2r, x3r, x4r, gr, w1ar, w1br, g1r, b1r, m1r, v1r,
             w2r, g2r, b2r, m2r, v2r, w3r, outr):
        xc = jnp.concatenate([x1r[0], x2r[0], x3r[0], x4r[0]], axis=1)
        h = _mm(xc, w1ar[...]) + _mm(gr[0], w1br[...])
        h = _bn_lrelu(h, g1r[...], b1r[...], m1r[...], v1r[...])
        h = _bn_lrelu(_mm(h, w2r[...]), g2r[...], b2r[...], m2r[...], v2r[...])
        outr[0] = _mm(h, w3r[...])

    ins = [pl.BlockSpec((1, blk, a.shape[2]), lambda b, n: (b, n, 0))
           for a in (x1, x2, x3, x4)]
    return pl.pallas_call(
        body,
        grid=(B, N // blk),
        in_specs=ins + [
            pl.BlockSpec((1, 1, xglob.shape[2]), lambda b, n: (b, 0, 0)),
            pl.BlockSpec(Wh1a.shape, lambda b, n: (0, 0)),
            pl.BlockSpec(Wh1b.shape, lambda b, n: (0, 0)),
            pl.BlockSpec((1, 512), lambda b, n: (0, 0)),
            pl.BlockSpec((1, 512), lambda b, n: (0, 0)),
            pl.BlockSpec((1, 512), lambda b, n: (0, 0)),
            pl.BlockSpec((1, 512), lambda b, n: (0, 0)),
            pl.BlockSpec(Wh2.shape, lambda b, n: (0, 0)),
            pl.BlockSpec((1, 256), lambda b, n: (0, 0)),
            pl.BlockSpec((1, 256), lambda b, n: (0, 0)),
            pl.BlockSpec((1, 256), lambda b, n: (0, 0)),
            pl.BlockSpec((1, 256), lambda b, n: (0, 0)),
            pl.BlockSpec(Wh3.shape, lambda b, n: (0, 0)),
        ],
        out_specs=pl.BlockSpec((1, blk, NC), lambda b, n: (b, n, 0)),
        out_shape=jax.ShapeDtypeStruct((B, N, NC), jnp.float32),
    )(x1, x2, x3, x4, xglob, Wh1a, Wh1b,
      gh1.reshape(1, -1), bh1.reshape(1, -1), mh1.reshape(1, -1),
      vh1.reshape(1, -1), Wh2, gh2.reshape(1, -1), bh2.reshape(1, -1),
      mh2.reshape(1, -1), vh2.reshape(1, -1), Wh3)


def _edge_layer(xp, W, g, b, m, v, C, cpn):
    R, Cp = xp.shape
    B, N = 4, 2048
    idx = _knn_call(xp.reshape(B, N, Cp))
    rows = _gather_rows_sc(xp, idx.reshape(32, (R * _K) // (32 * 80), 80))
    return _edgeconv_call(xp, rows, W, g, b, m, v, C, cpn)


def kernel(x, W1, g1, b1, m1, v1, W2, g2, b2, m2, v2, W3, g3, b3, m3, v3,
           W4, g4, b4, m4, v4, W5, g5, b5, m5, v5, Wh1, gh1, bh1, mh1, vh1,
           Wh2, gh2, bh2, mh2, vh2, Wh3):
    B, N, F = x.shape
    R = B * N
    xp0 = jnp.pad(x.reshape(R, F), ((0, 0), (0, 128 - F)))

    x1, x1p = _edge_layer(xp0, W1, g1, b1, m1, v1, F, 128)
    x2, x2p = _edge_layer(x1p, W2, g2, b2, m2, v2, 64, 128)
    x3, x3p = _edge_layer(x2p, W3, g3, b3, m3, v3, 64, 128)
    x4, _ = _edge_layer(x3p, W4, g4, b4, m4, v4, 128, 256)

    x1 = x1.reshape(B, N, 64)
    x2 = x2.reshape(B, N, 64)
    x3 = x3.reshape(B, N, 128)
    x4 = x4.reshape(B, N, 256)

    xglob = _head_global(x1, x2, x3, x4, W5, g5, b5, m5, v5)
    return _head_seg(x1, x2, x3, x4, xglob, Wh1[:, :512], Wh1[:, 512:],
                     gh1, bh1, mh1, vh1, Wh2, gh2, bh2, mh2, vh2, Wh3)
